# Initial kernel scaffold; baseline (speedup 1.0000x reference)
import functools

import jax
import jax.numpy as jnp
from jax import lax
from jax.experimental import pallas as pl
from jax.experimental.pallas import tpu as pltpu
from jax.experimental.pallas import tpu_sc as plsc

N = 10000
E = 320000
KK = 25
NUM_G = 64
F_IN = 128
F1 = 32
F2 = 64

NC = 2
NS = 16
NTILE = NC * NS
E_PER_TILE = E // NTILE
BLK = 80
NBLK = E_PER_TILE // BLK
ROWS_PER_TILE = N // NS
ZROWS = 125

_mesh = plsc.VectorSubcoreMesh(core_axis_name="c", subcore_axis_name="s")


def _f32(shape):
    return jax.ShapeDtypeStruct(shape, jnp.float32)


def _sc_edge_kernel(fo, with_deg):
    nch = fo // 16

    scratch = [
        pltpu.VMEM((BLK,), jnp.int32),
        pltpu.VMEM((BLK,), jnp.int32),
        pltpu.VMEM((BLK,), jnp.float32),
        pltpu.VMEM((BLK,), jnp.float32),
        pltpu.VMEM((BLK,), jnp.int32),
        pltpu.VMEM((BLK,), jnp.int32),
        pltpu.VMEM((BLK,), jnp.int32),
        pltpu.VMEM((BLK,), jnp.int32),
        pltpu.VMEM((BLK,), jnp.float32),
        pltpu.VMEM((BLK,), jnp.float32),
        pltpu.VMEM((BLK,), jnp.float32),
        pltpu.VMEM((BLK,), jnp.float32),
        pltpu.VMEM((BLK, fo), jnp.float32),
        pltpu.VMEM((BLK, fo), jnp.float32),
        pltpu.VMEM((BLK, fo), jnp.float32),
        pltpu.VMEM((BLK, fo), jnp.float32),
        pltpu.VMEM((BLK, fo), jnp.float32),
        pltpu.VMEM((ZROWS, fo), jnp.float32),
        pltpu.VMEM_SHARED((N, fo), jnp.float32),
    ]
    out_type = [_f32((NC, N, fo))]
    if with_deg:
        scratch += [
            pltpu.VMEM((BLK, 16), jnp.float32),
            pltpu.VMEM((ZROWS, 16), jnp.float32),
            pltpu.VMEM_SHARED((N, 16), jnp.float32),
        ]
        out_type.append(_f32((NC, N, 16)))

    def body(y_hbm, src_hbm, dst_hbm, p0_hbm, p1_hbm, *refs):
        if with_deg:
            (msg_out, deg_out,
             srcv, dstv, p0v, p1v,
             idx0, idx1, idx2, idx3, w0, w1, w2, w3,
             rows0, rows1, rows2, rows3, accr, zbuf, accsh,
             onesb, zdeg, degsh) = refs
        else:
            (msg_out,
             srcv, dstv, p0v, p1v,
             idx0, idx1, idx2, idx3, w0, w1, w2, w3,
             rows0, rows1, rows2, rows3, accr, zbuf, accsh) = refs

        c = lax.axis_index("c")
        s = lax.axis_index("s")
        gid = c * NS + s
        row0 = s * ROWS_PER_TILE

        zv = jnp.zeros((16,), jnp.float32)

        @pl.loop(0, ZROWS)
        def _(r):
            for ch in range(nch):
                zbuf[r, pl.ds(ch * 16, 16)] = zv
            if with_deg:
                zdeg[r, pl.ds(0, 16)] = zv

        if with_deg:
            ov = jnp.ones((16,), jnp.float32)

            @pl.loop(0, BLK)
            def _(r):
                onesb[r, pl.ds(0, 16)] = ov

        @pl.loop(0, 5)
        def _(j):
            pltpu.sync_copy(zbuf, accsh.at[pl.ds(row0 + j * ZROWS, ZROWS)])
            if with_deg:
                pltpu.sync_copy(zdeg, degsh.at[pl.ds(row0 + j * ZROWS, ZROWS)])

        plsc.subcore_barrier()

        @pl.loop(0, NBLK)
        def _(blk):
            base = gid * E_PER_TILE + blk * BLK
            pltpu.sync_copy(src_hbm.at[pl.ds(base, BLK)], srcv)
            pltpu.sync_copy(dst_hbm.at[pl.ds(base, BLK)], dstv)
            pltpu.sync_copy(p0_hbm.at[pl.ds(base, BLK)], p0v)
            pltpu.sync_copy(p1_hbm.at[pl.ds(base, BLK)], p1v)

            @pl.loop(0, BLK // 16)
            def _(g):
                sl = pl.ds(g * 16, 16)
                v0 = p0v[sl] * 4.0
                v1 = p1v[sl] * 4.0
                i0 = v0.astype(jnp.int32)
                i1 = v1.astype(jnp.int32)
                fr0 = v0 - i0.astype(jnp.float32)
                fr1 = v1 - i1.astype(jnp.float32)
                base_idx = srcv[sl] * KK
                i0b = jnp.minimum(i0 + 1, 4)
                i1b = jnp.minimum(i1 + 1, 4)
                idx0[sl] = base_idx + i0 + 5 * i1
                idx1[sl] = base_idx + i0b + 5 * i1
                idx2[sl] = base_idx + i0 + 5 * i1b
                idx3[sl] = base_idx + i0b + 5 * i1b
                g0 = 1.0 - fr0
                g1 = 1.0 - fr1
                w0[sl] = g0 * g1
                w1[sl] = fr0 * g1
                w2[sl] = g0 * fr1
                w3[sl] = fr0 * fr1

            pltpu.sync_copy(y_hbm.at[idx0], rows0)
            pltpu.sync_copy(y_hbm.at[idx1], rows1)
            pltpu.sync_copy(y_hbm.at[idx2], rows2)
            pltpu.sync_copy(y_hbm.at[idx3], rows3)

            @pl.loop(0, BLK // 16)
            def _(g):
                for b in range(16):
                    rb = g * 16 + b
                    spl = jnp.full((16,), rb, jnp.int32)
                    s0 = plsc.load_gather(w0, [spl])
                    s1 = plsc.load_gather(w1, [spl])
                    s2 = plsc.load_gather(w2, [spl])
                    s3 = plsc.load_gather(w3, [spl])
                    for ch in range(nch):
                        sl = pl.ds(ch * 16, 16)
                        accr[rb, sl] = (rows0[rb, sl] * s0 + rows1[rb, sl] * s1
                                        + rows2[rb, sl] * s2 + rows3[rb, sl] * s3)

            pltpu.sync_copy(accr, accsh.at[dstv], add=True)
            if with_deg:
                pltpu.sync_copy(onesb, degsh.at[dstv], add=True)

        plsc.subcore_barrier()

        pltpu.sync_copy(accsh.at[pl.ds(row0, ROWS_PER_TILE)],
                        msg_out.at[c, pl.ds(row0, ROWS_PER_TILE)])
        if with_deg:
            pltpu.sync_copy(degsh.at[pl.ds(row0, ROWS_PER_TILE)],
                            deg_out.at[c, pl.ds(row0, ROWS_PER_TILE)])

    return pl.kernel(body, out_type=tuple(out_type), mesh=_mesh,
                     scratch_types=scratch)


_sc_layer1 = _sc_edge_kernel(F1, with_deg=True)
_sc_layer2 = _sc_edge_kernel(F2, with_deg=False)


ROWB = 1000
NROWB = N // ROWB


def _mm_body(x_ref, w_ref, o_ref):
    o_ref[...] = jnp.dot(x_ref[...], w_ref[...],
                         preferred_element_type=jnp.float32)


_tc_a = pl.pallas_call(
    _mm_body,
    grid=(NROWB,),
    in_specs=[
        pl.BlockSpec((ROWB, F_IN), lambda i: (i, 0)),
        pl.BlockSpec((F_IN, KK * F1), lambda i: (0, 0)),
    ],
    out_specs=pl.BlockSpec((ROWB, KK * F1), lambda i: (i, 0)),
    out_shape=_f32((N, KK * F1)),
)


def _elu(v):
    return jnp.where(v > 0, v, jnp.expm1(jnp.minimum(v, 0.0)))


def _tc_b_body(x_ref, r1_ref, b1_ref, mp_ref, dp_ref, w2_ref, h1_ref, y2_ref):
    msg = mp_ref[0] + mp_ref[1]
    dsum = dp_ref[0] + dp_ref[1]
    deg = jnp.maximum(dsum[:, 0:1], 1.0)
    pre = msg / deg + jnp.dot(x_ref[...], r1_ref[...],
                              preferred_element_type=jnp.float32) + b1_ref[...]
    h1 = _elu(pre)
    h1_ref[...] = h1
    y2_ref[...] = jnp.dot(h1, w2_ref[...], preferred_element_type=jnp.float32)


_tc_b = pl.pallas_call(
    _tc_b_body,
    grid=(NROWB,),
    in_specs=[
        pl.BlockSpec((ROWB, F_IN), lambda i: (i, 0)),
        pl.BlockSpec((F_IN, F1), lambda i: (0, 0)),
        pl.BlockSpec((1, F1), lambda i: (0, 0)),
        pl.BlockSpec((NC, ROWB, F1), lambda i: (0, i, 0)),
        pl.BlockSpec((NC, ROWB, 16), lambda i: (0, i, 0)),
        pl.BlockSpec((F1, KK * F2), lambda i: (0, 0)),
    ],
    out_specs=[
        pl.BlockSpec((ROWB, F1), lambda i: (i, 0)),
        pl.BlockSpec((ROWB, KK * F2), lambda i: (i, 0)),
    ],
    out_shape=[_f32((N, F1)), _f32((N, KK * F2))],
)


def _tc_c_body(h1_ref, r2_ref, b2_ref, mp_ref, dp_ref, batch_ref,
               f1w_ref, f1b_ref, f2w_ref, f2b_ref, o_ref, pooled, cnts):
    i = pl.program_id(0)

    @pl.when(i == 0)
    def _():
        pooled[...] = jnp.zeros_like(pooled)
        cnts[...] = jnp.zeros_like(cnts)

    msg = mp_ref[0] + mp_ref[1]
    dsum = dp_ref[0] + dp_ref[1]
    deg = jnp.maximum(dsum[:, 0:1], 1.0)
    pre = msg / deg + jnp.dot(h1_ref[...], r2_ref[...],
                              preferred_element_type=jnp.float32) + b2_ref[...]
    h2 = _elu(pre)
    bb = batch_ref[0, 0, :]
    gids = lax.broadcasted_iota(jnp.int32, (NUM_G, ROWB), 0)
    onehot = (bb[None, :] == gids).astype(jnp.float32)
    pooled[...] += jnp.dot(onehot, h2, preferred_element_type=jnp.float32)
    cnts[...] += jnp.sum(onehot, axis=1, keepdims=True)

    @pl.when(i == NROWB - 1)
    def _():
        pool = pooled[...] / jnp.maximum(cnts[...][:, 0:1], 1.0)
        z = _elu(jnp.dot(pool, f1w_ref[...],
                         preferred_element_type=jnp.float32) + f1b_ref[...])
        o = jnp.dot(z, f2w_ref[...],
                    preferred_element_type=jnp.float32) + f2b_ref[...]
        mx = jnp.max(o, axis=1, keepdims=True)
        o_ref[...] = o - mx - jnp.log(
            jnp.sum(jnp.exp(o - mx), axis=1, keepdims=True))


_tc_c = pl.pallas_call(
    _tc_c_body,
    grid=(NROWB,),
    in_specs=[
        pl.BlockSpec((ROWB, F1), lambda i: (i, 0)),
        pl.BlockSpec((F1, F2), lambda i: (0, 0)),
        pl.BlockSpec((1, F2), lambda i: (0, 0)),
        pl.BlockSpec((NC, ROWB, F2), lambda i: (0, i, 0)),
        pl.BlockSpec((NC, ROWB, 16), lambda i: (0, i, 0)),
        pl.BlockSpec((1, 1, ROWB), lambda i: (i, 0, 0)),
        pl.BlockSpec((F2, 128), lambda i: (0, 0)),
        pl.BlockSpec((1, 128), lambda i: (0, 0)),
        pl.BlockSpec((128, 10), lambda i: (0, 0)),
        pl.BlockSpec((1, 10), lambda i: (0, 0)),
    ],
    out_specs=pl.BlockSpec((NUM_G, 10), lambda i: (0, 0)),
    out_shape=_f32((NUM_G, 10)),
    scratch_shapes=[
        pltpu.VMEM((NUM_G, F2), jnp.float32),
        pltpu.VMEM((NUM_G, 128), jnp.float32),
    ],
)


def kernel(x, edge_index, edge_attr, batch, W1, R1, b1, W2, R2, b2,
           fc1_w, fc1_b, fc2_w, fc2_b):
    src = edge_index[0]
    dst = edge_index[1]
    p0 = edge_attr[:, 0]
    p1 = edge_attr[:, 1]
    w1cat = jnp.transpose(W1, (1, 0, 2)).reshape(F_IN, KK * F1)
    w2cat = jnp.transpose(W2, (1, 0, 2)).reshape(F1, KK * F2)

    y1 = _tc_a(x, w1cat).reshape(N * KK, F1)
    mp1, dp = _sc_layer1(y1, src, dst, p0, p1)
    h1, y2 = _tc_b(x, R1, b1.reshape(1, F1), mp1, dp, w2cat)
    (mp2,) = _sc_layer2(y2.reshape(N * KK, F2), src, dst, p0, p1)
    out = _tc_c(h1, R2, b2.reshape(1, F2), mp2, dp,
                batch.reshape(NROWB, 1, ROWB), fc1_w, fc1_b.reshape(1, 128),
                fc2_w, fc2_b.reshape(1, 10))
    return out


# speedup vs baseline: 6.9636x; 6.9636x over previous
import dataclasses
import functools

import jax
import jax.numpy as jnp
from jax import lax
from jax.experimental import pallas as pl
from jax.experimental.pallas import tpu as pltpu
from jax.experimental.pallas import tpu_sc as plsc

N = 10000
E = 320000
KK = 25
NUM_G = 64
F_IN = 128
F1 = 32
F2 = 64

NC = 2
NS = 16
NTILE = NC * NS
E_PER_TILE = E // NTILE
BLK = 80
NBLK = E_PER_TILE // BLK
RCHUNK = 80
NRCHUNK = N // RCHUNK

_mesh = plsc.VectorSubcoreMesh(core_axis_name="c", subcore_axis_name="s")


def _f32(shape):
    return jax.ShapeDtypeStruct(shape, jnp.float32)


BLKE = 128
NEB = E // BLKE


def _sc_edge_kernel(fo, ngather, with_deg):
    nch = fo // 16

    CH = BLKE // ngather
    scratch = [
        pltpu.VMEM((1, BLKE), jnp.int32),
        pltpu.VMEM((BLKE,), jnp.float32),
        pltpu.VMEM((BLKE,), jnp.float32),
        pltpu.VMEM((BLKE,), jnp.float32),
        pltpu.VMEM((BLKE,), jnp.float32),
        pltpu.VMEM((CH, 128), jnp.float32),
        pltpu.VMEM((CH, 128), jnp.float32),
        pltpu.VMEM((16, 128), jnp.float32),
        pltpu.VMEM_SHARED((N, 128), jnp.float32),
    ]
    if ngather == 2:
        scratch += [
            pltpu.VMEM((1, BLKE), jnp.int32),
            pltpu.VMEM((CH, 128), jnp.float32),
            pltpu.VMEM((1, CH), jnp.int32),
        ]
    out_type = _f32((NC, N, 128))
    NZCH = N // 16

    def body(y_hbm, src_hbm, dst_hbm, p0_hbm, p1_hbm, msg_out, *refs):
        if ngather == 2:
            (idx0, w0, w1, w2, w3, rows0, accr, zbuf, accsh,
             idx1, rows1, dsth) = refs
        else:
            (idx0, w0, w1, w2, w3, rows0, accr, zbuf, accsh) = refs
            idx1 = rows1 = dsth = None

        c = lax.axis_index("c")
        s = lax.axis_index("s")

        zv = jnp.zeros((16,), jnp.float32)

        @pl.loop(0, 16)
        def _(r):
            for ch in range(8):
                zbuf[r, pl.ds(ch * 16, 16)] = zv

        ov = jnp.ones((16,), jnp.float32)

        @pl.loop(0, CH)
        def _(r):
            for ch in range(8):
                col = ch * 16
                if with_deg and col == F1:
                    accr[r, pl.ds(col, 16)] = ov
                elif col >= fo:
                    accr[r, pl.ds(col, 16)] = zv

        for j in range((NZCH + NS - 1) // NS):
            k = j * NS + s

            @pl.when(k < NZCH)
            def _():
                pltpu.sync_copy(zbuf, accsh.at[pl.ds(k * 16, 16)])

        plsc.subcore_barrier()

        if ngather == 1:
            taps = [(0, 0), (0, fo), (0, 2 * fo), (0, 3 * fo)]
        else:
            taps = [(0, 0), (0, fo), (1, 0), (1, fo)]

        dn = lax.GatherDimensionNumbers(
            offset_dims=(), collapsed_slice_dims=(0,), start_index_map=(0,))

        def _splat(vec, b):
            return lax.gather(
                vec, jnp.full((16, 1), b, jnp.int32), dn, slice_sizes=(1,),
                mode=lax.GatherScatterMode.PROMISE_IN_BOUNDS)

        def _combine(off):
            bufs = (rows0, rows1)

            @pl.loop(0, CH // 16)
            def _(g):
                sl = pl.ds(off + g * 16, 16)
                wr = (w0[sl], w1[sl], w2[sl], w3[sl])
                for b in range(16):
                    rb = g * 16 + b
                    svec = tuple(_splat(wr[cc], b) for cc in range(4))
                    for ch in range(nch):
                        acc = None
                        for cc, (bi, coff) in enumerate(taps):
                            term = (bufs[bi][rb, pl.ds(coff + ch * 16, 16)]
                                    * svec[cc])
                            acc = term if acc is None else acc + term
                        accr[rb, pl.ds(ch * 16, 16)] = acc

        def pbody(src_v, dst_v, p0_v, p1_v):
            for g in range(BLKE // 16):
                sl = pl.ds(g * 16, 16)
                v0 = p0_v[0, sl] * 4.0
                v1 = p1_v[0, sl] * 4.0
                i0 = v0.astype(jnp.int32)
                i1 = v1.astype(jnp.int32)
                fr0 = v0 - i0.astype(jnp.float32)
                fr1 = v1 - i1.astype(jnp.float32)
                q = src_v[0, sl] * KK + i0 + 5 * i1
                idx0[0, sl] = q
                if ngather == 2:
                    idx1[0, sl] = q + 5
                g0 = 1.0 - fr0
                g1 = 1.0 - fr1
                w0[sl] = g0 * g1
                w1[sl] = fr0 * g1
                w2[sl] = g0 * fr1
                w3[sl] = fr0 * fr1

            if ngather == 1:
                pltpu.sync_copy(y_hbm.at[idx0.at[0]], rows0)
                _combine(0)
                pltpu.sync_copy(accr, accsh.at[dst_v.at[0]], add=True)
            else:
                for h in range(2):
                    off = h * CH
                    for g2 in range(CH // 16):
                        sl16 = pl.ds(g2 * 16, 16)
                        dsth[0, sl16] = dst_v[0, pl.ds(off + g2 * 16, 16)]
                    pltpu.sync_copy(y_hbm.at[idx0.at[0, pl.ds(off, CH)]],
                                    rows0)
                    pltpu.sync_copy(y_hbm.at[idx1.at[0, pl.ds(off, CH)]],
                                    rows1)
                    _combine(off)
                    pltpu.sync_copy(accr, accsh.at[dsth.at[0]], add=True)

        pltpu.emit_pipeline(
            pbody,
            grid=(NEB,),
            in_specs=[pl.BlockSpec((1, BLKE), lambda i: (i, 0))] * 4,
            out_specs=[],
            core_axis_name=("c", "s"),
            dimension_semantics=(pltpu.PARALLEL,),
        )(src_hbm, dst_hbm, p0_hbm, p1_hbm)

        plsc.subcore_barrier()

        for j in range((NRCHUNK + NS - 1) // NS):
            k = j * NS + s

            @pl.when(k < NRCHUNK)
            def _():
                pltpu.sync_copy(accsh.at[pl.ds(k * RCHUNK, RCHUNK)],
                                msg_out.at[c, pl.ds(k * RCHUNK, RCHUNK)])

    cp = pltpu.CompilerParams(use_tc_tiling_on_sc=False)
    return pl.kernel(body, out_type=out_type, mesh=_mesh,
                     scratch_types=scratch, compiler_params=cp)


_sc_layer1 = _sc_edge_kernel(F1, ngather=1, with_deg=True)
_sc_layer2 = _sc_edge_kernel(F2, ngather=2, with_deg=False)

PACKW = KK * 128


def _pack_taps(W, offsets):
    z = jnp.zeros_like(W[0])
    cols = [W[t + o] if t + o < KK else z
            for t in range(KK) for o in offsets]
    return jnp.concatenate(cols, axis=1)


ROWB = 1000
NROWB = N // ROWB


def _mm_body(x_ref, w_ref, o_ref):
    o_ref[...] = jnp.dot(x_ref[...], w_ref[...],
                         preferred_element_type=jnp.float32)


_tc_a = pl.pallas_call(
    _mm_body,
    grid=(NROWB,),
    in_specs=[
        pl.BlockSpec((ROWB, F_IN), lambda i: (i, 0)),
        pl.BlockSpec((F_IN, PACKW), lambda i: (0, 0)),
    ],
    out_specs=pl.BlockSpec((ROWB, PACKW), lambda i: (i, 0)),
    out_shape=_f32((N, PACKW)),
)


def _elu(v):
    return jnp.where(v > 0, v, jnp.exp(jnp.minimum(v, 0.0)) - 1.0)


def _tc_b_body(x_ref, r1_ref, b1_ref, mp_ref, w2_ref, h1_ref, y2_ref):
    mps = mp_ref[0] + mp_ref[1]
    msg = mps[:, 0:F1]
    deg = jnp.maximum(mps[:, F1:F1 + 1], 1.0)
    pre = msg / deg + jnp.dot(x_ref[...], r1_ref[...],
                              preferred_element_type=jnp.float32) + b1_ref[...]
    h1 = _elu(pre)
    h1_ref[...] = h1
    y2_ref[...] = jnp.dot(h1, w2_ref[...], preferred_element_type=jnp.float32)


_tc_b = pl.pallas_call(
    _tc_b_body,
    grid=(NROWB,),
    in_specs=[
        pl.BlockSpec((ROWB, F_IN), lambda i: (i, 0)),
        pl.BlockSpec((F_IN, F1), lambda i: (0, 0)),
        pl.BlockSpec((1, F1), lambda i: (0, 0)),
        pl.BlockSpec((NC, ROWB, 128), lambda i: (0, i, 0)),
        pl.BlockSpec((F1, PACKW), lambda i: (0, 0)),
    ],
    out_specs=[
        pl.BlockSpec((ROWB, F1), lambda i: (i, 0)),
        pl.BlockSpec((ROWB, PACKW), lambda i: (i, 0)),
    ],
    out_shape=[_f32((N, F1)), _f32((N, PACKW))],
)


def _tc_c_body(h1_ref, r2_ref, b2_ref, mp_ref, dp_ref, batch_ref,
               f1w_ref, f1b_ref, f2w_ref, f2b_ref, o_ref, pooled, cnts):
    i = pl.program_id(0)

    @pl.when(i == 0)
    def _():
        pooled[...] = jnp.zeros_like(pooled)
        cnts[...] = jnp.zeros_like(cnts)

    msg = (mp_ref[0] + mp_ref[1])[:, 0:F2]
    dsum = dp_ref[0] + dp_ref[1]
    deg = jnp.maximum(dsum[:, F1:F1 + 1], 1.0)
    pre = msg / deg + jnp.dot(h1_ref[...], r2_ref[...],
                              preferred_element_type=jnp.float32) + b2_ref[...]
    h2 = _elu(pre)
    bb = batch_ref[0, 0, :]
    gids = lax.broadcasted_iota(jnp.int32, (NUM_G, ROWB), 0)
    onehot = (bb[None, :] == gids).astype(jnp.float32)
    pooled[...] += jnp.dot(onehot, h2, preferred_element_type=jnp.float32)
    cnts[...] += jnp.sum(onehot, axis=1, keepdims=True)

    @pl.when(i == NROWB - 1)
    def _():
        pool = pooled[...] / jnp.maximum(cnts[...][:, 0:1], 1.0)
        z = _elu(jnp.dot(pool, f1w_ref[...],
                         preferred_element_type=jnp.float32) + f1b_ref[...])
        o = jnp.dot(z, f2w_ref[...],
                    preferred_element_type=jnp.float32) + f2b_ref[...]
        mx = jnp.max(o, axis=1, keepdims=True)
        o_ref[...] = o - mx - jnp.log(
            jnp.sum(jnp.exp(o - mx), axis=1, keepdims=True))


_tc_c = pl.pallas_call(
    _tc_c_body,
    grid=(NROWB,),
    in_specs=[
        pl.BlockSpec((ROWB, F1), lambda i: (i, 0)),
        pl.BlockSpec((F1, F2), lambda i: (0, 0)),
        pl.BlockSpec((1, F2), lambda i: (0, 0)),
        pl.BlockSpec((NC, ROWB, 128), lambda i: (0, i, 0)),
        pl.BlockSpec((NC, ROWB, 128), lambda i: (0, i, 0)),
        pl.BlockSpec((1, 1, ROWB), lambda i: (i, 0, 0)),
        pl.BlockSpec((F2, 128), lambda i: (0, 0)),
        pl.BlockSpec((1, 128), lambda i: (0, 0)),
        pl.BlockSpec((128, 10), lambda i: (0, 0)),
        pl.BlockSpec((1, 10), lambda i: (0, 0)),
    ],
    out_specs=pl.BlockSpec((NUM_G, 10), lambda i: (0, 0)),
    out_shape=_f32((NUM_G, 10)),
    scratch_shapes=[
        pltpu.VMEM((NUM_G, F2), jnp.float32),
        pltpu.VMEM((NUM_G, 128), jnp.float32),
    ],
)


def kernel(x, edge_index, edge_attr, batch, W1, R1, b1, W2, R2, b2,
           fc1_w, fc1_b, fc2_w, fc2_b):
    src = edge_index[0].reshape(NEB, BLKE)
    dst = edge_index[1].reshape(NEB, BLKE)
    p0 = edge_attr[:, 0].reshape(NEB, BLKE)
    p1 = edge_attr[:, 1].reshape(NEB, BLKE)
    w1big = _pack_taps(W1, (0, 1, 5, 6))
    w2big = _pack_taps(W2, (0, 1))

    y1 = _tc_a(x, w1big).reshape(N * KK, 4 * F1)
    mp1 = _sc_layer1(y1, src, dst, p0, p1)
    h1, y2 = _tc_b(x, R1, b1.reshape(1, F1), mp1, w2big)
    mp2 = _sc_layer2(y2.reshape(N * KK, 2 * F2), src, dst, p0, p1)
    out = _tc_c(h1, R2, b2.reshape(1, F2), mp2, mp1,
                batch.reshape(NROWB, 1, ROWB), fc1_w, fc1_b.reshape(1, 128),
                fc2_w, fc2_b.reshape(1, 10))
    return out


# speedup vs baseline: 7.9725x; 1.1449x over previous
import dataclasses
import functools

import jax
import jax.numpy as jnp
from jax import lax
from jax.experimental import pallas as pl
from jax.experimental.pallas import tpu as pltpu
from jax.experimental.pallas import tpu_sc as plsc

N = 10000
E = 320000
KK = 25
NUM_G = 64
F_IN = 128
F1 = 32
F2 = 64

NC = 2
NS = 16
NTILE = NC * NS
E_PER_TILE = E // NTILE
BLK = 80
NBLK = E_PER_TILE // BLK
RCHUNK = 80
NRCHUNK = N // RCHUNK

_mesh = plsc.VectorSubcoreMesh(core_axis_name="c", subcore_axis_name="s")


def _f32(shape):
    return jax.ShapeDtypeStruct(shape, jnp.float32)


BLKE = 128
NEB = E // BLKE


def _sc_edge_kernel(fo, ngather, with_deg):
    nch = fo // 16

    CH = 64
    scratch = [
        pltpu.VMEM((1, BLKE), jnp.int32),
        pltpu.VMEM((BLKE,), jnp.float32),
        pltpu.VMEM((BLKE,), jnp.float32),
        pltpu.VMEM((BLKE,), jnp.float32),
        pltpu.VMEM((BLKE,), jnp.float32),
        pltpu.VMEM((CH, 128), jnp.float32),
        pltpu.VMEM((CH, 128), jnp.float32),
        pltpu.VMEM((CH, 128), jnp.float32),
        pltpu.VMEM((16, 128), jnp.float32),
        pltpu.VMEM((1, CH), jnp.int32),
        pltpu.VMEM_SHARED((N, 128), jnp.float32),
        pltpu.SemaphoreType.DMA,
        pltpu.SemaphoreType.DMA,
    ]
    if ngather == 2:
        scratch += [
            pltpu.VMEM((1, BLKE), jnp.int32),
            pltpu.VMEM((CH, 128), jnp.float32),
            pltpu.VMEM((CH, 128), jnp.float32),
        ]
    out_type = _f32((NC, N, 128))
    NZCH = N // 16

    def body(y_hbm, src_hbm, dst_hbm, p0_hbm, p1_hbm, msg_out, *refs):
        if ngather == 2:
            (idx0, w0, w1, w2, w3, rowsA0, rowsA1, accr, zbuf, dsth, accsh,
             sem0, sem1, idx1, rowsB0, rowsB1) = refs
        else:
            (idx0, w0, w1, w2, w3, rowsA0, rowsA1, accr, zbuf, dsth, accsh,
             sem0, sem1) = refs
            idx1 = rowsB0 = rowsB1 = None

        c = lax.axis_index("c")
        s = lax.axis_index("s")

        zv = jnp.zeros((16,), jnp.float32)

        @pl.loop(0, 16)
        def _(r):
            for ch in range(8):
                zbuf[r, pl.ds(ch * 16, 16)] = zv

        ov = jnp.ones((16,), jnp.float32)

        @pl.loop(0, CH)
        def _(r):
            for ch in range(8):
                col = ch * 16
                if with_deg and col == F1:
                    accr[r, pl.ds(col, 16)] = ov
                elif col >= fo:
                    accr[r, pl.ds(col, 16)] = zv

        for j in range((NZCH + NS - 1) // NS):
            k = j * NS + s

            @pl.when(k < NZCH)
            def _():
                pltpu.sync_copy(zbuf, accsh.at[pl.ds(k * 16, 16)])

        plsc.subcore_barrier()

        if ngather == 1:
            taps = [(0, 0), (0, fo), (0, 2 * fo), (0, 3 * fo)]
        else:
            taps = [(0, 0), (0, fo), (1, 0), (1, fo)]

        dn = lax.GatherDimensionNumbers(
            offset_dims=(), collapsed_slice_dims=(0,), start_index_map=(0,))

        def _splat(vec, b):
            return lax.gather(
                vec, jnp.full((16, 1), b, jnp.int32), dn, slice_sizes=(1,),
                mode=lax.GatherScatterMode.PROMISE_IN_BOUNDS)

        def _combine(off, bufs):
            @pl.loop(0, CH // 16)
            def _(g):
                sl = pl.ds(off + g * 16, 16)
                wr = (w0[sl], w1[sl], w2[sl], w3[sl])
                for b in range(16):
                    rb = g * 16 + b
                    svec = tuple(_splat(wr[cc], b) for cc in range(4))
                    for ch in range(nch):
                        acc = None
                        for cc, (bi, coff) in enumerate(taps):
                            term = (bufs[bi][rb, pl.ds(coff + ch * 16, 16)]
                                    * svec[cc])
                            acc = term if acc is None else acc + term
                        accr[rb, pl.ds(ch * 16, 16)] = acc

        def pbody(src_v, dst_v, p0_v, p1_v):
            for g in range(BLKE // 16):
                sl = pl.ds(g * 16, 16)
                v0 = p0_v[0, sl] * 4.0
                v1 = p1_v[0, sl] * 4.0
                i0 = v0.astype(jnp.int32)
                i1 = v1.astype(jnp.int32)
                fr0 = v0 - i0.astype(jnp.float32)
                fr1 = v1 - i1.astype(jnp.float32)
                q = src_v[0, sl] * KK + i0 + 5 * i1
                idx0[0, sl] = q
                if ngather == 2:
                    idx1[0, sl] = q + 5
                g0 = 1.0 - fr0
                g1 = 1.0 - fr1
                w0[sl] = g0 * g1
                w1[sl] = fr0 * g1
                w2[sl] = g0 * fr1
                w3[sl] = fr0 * fr1

            sems = (sem0, sem1)
            arows = (rowsA0, rowsA1)
            brows = (rowsB0, rowsB1)
            handles = []
            for h in range(2):
                hh = [pltpu.async_copy(
                    y_hbm.at[idx0.at[0, pl.ds(h * CH, CH)]], arows[h],
                    sems[h])]
                if ngather == 2:
                    hh.append(pltpu.async_copy(
                        y_hbm.at[idx1.at[0, pl.ds(h * CH, CH)]], brows[h],
                        sems[h]))
                handles.append(hh)

            for h in range(2):
                off = h * CH
                for g2 in range(CH // 16):
                    sl16 = pl.ds(g2 * 16, 16)
                    dsth[0, sl16] = dst_v[0, pl.ds(off + g2 * 16, 16)]
                for hdl in handles[h]:
                    hdl.wait()
                _combine(off, (arows[h], brows[h]))
                pltpu.sync_copy(accr, accsh.at[dsth.at[0]], add=True)

        pltpu.emit_pipeline(
            pbody,
            grid=(NEB,),
            in_specs=[pl.BlockSpec((1, BLKE), lambda i: (i, 0))] * 4,
            out_specs=[],
            core_axis_name=("c", "s"),
            dimension_semantics=(pltpu.PARALLEL,),
        )(src_hbm, dst_hbm, p0_hbm, p1_hbm)

        plsc.subcore_barrier()

        for j in range((NRCHUNK + NS - 1) // NS):
            k = j * NS + s

            @pl.when(k < NRCHUNK)
            def _():
                pltpu.sync_copy(accsh.at[pl.ds(k * RCHUNK, RCHUNK)],
                                msg_out.at[c, pl.ds(k * RCHUNK, RCHUNK)])

    cp = pltpu.CompilerParams(use_tc_tiling_on_sc=False)
    return pl.kernel(body, out_type=out_type, mesh=_mesh,
                     scratch_types=scratch, compiler_params=cp)


_sc_layer1 = _sc_edge_kernel(F1, ngather=1, with_deg=True)
_sc_layer2 = _sc_edge_kernel(F2, ngather=2, with_deg=False)

PACKW = KK * 128


def _pack_taps(W, offsets):
    z = jnp.zeros_like(W[0])
    cols = [W[t + o] if t + o < KK else z
            for t in range(KK) for o in offsets]
    return jnp.concatenate(cols, axis=1)


ROWB = 1000
NROWB = N // ROWB


def _mm_body(x_ref, w_ref, o_ref):
    o_ref[...] = jnp.dot(x_ref[...], w_ref[...],
                         preferred_element_type=jnp.float32)


_tc_a = pl.pallas_call(
    _mm_body,
    grid=(NROWB,),
    in_specs=[
        pl.BlockSpec((ROWB, F_IN), lambda i: (i, 0)),
        pl.BlockSpec((F_IN, PACKW), lambda i: (0, 0)),
    ],
    out_specs=pl.BlockSpec((ROWB, PACKW), lambda i: (i, 0)),
    out_shape=_f32((N, PACKW)),
)


def _elu(v):
    return jnp.where(v > 0, v, jnp.exp(jnp.minimum(v, 0.0)) - 1.0)


def _tc_b_body(x_ref, r1_ref, b1_ref, mp_ref, w2_ref, h1_ref, y2_ref):
    mps = mp_ref[0] + mp_ref[1]
    msg = mps[:, 0:F1]
    deg = jnp.maximum(mps[:, F1:F1 + 1], 1.0)
    pre = msg / deg + jnp.dot(x_ref[...], r1_ref[...],
                              preferred_element_type=jnp.float32) + b1_ref[...]
    h1 = _elu(pre)
    h1_ref[...] = h1
    y2_ref[...] = jnp.dot(h1, w2_ref[...], preferred_element_type=jnp.float32)


_tc_b = pl.pallas_call(
    _tc_b_body,
    grid=(NROWB,),
    in_specs=[
        pl.BlockSpec((ROWB, F_IN), lambda i: (i, 0)),
        pl.BlockSpec((F_IN, F1), lambda i: (0, 0)),
        pl.BlockSpec((1, F1), lambda i: (0, 0)),
        pl.BlockSpec((NC, ROWB, 128), lambda i: (0, i, 0)),
        pl.BlockSpec((F1, PACKW), lambda i: (0, 0)),
    ],
    out_specs=[
        pl.BlockSpec((ROWB, F1), lambda i: (i, 0)),
        pl.BlockSpec((ROWB, PACKW), lambda i: (i, 0)),
    ],
    out_shape=[_f32((N, F1)), _f32((N, PACKW))],
)


def _tc_c_body(h1_ref, r2_ref, b2_ref, mp_ref, dp_ref, batch_ref,
               f1w_ref, f1b_ref, f2w_ref, f2b_ref, o_ref, pooled, cnts):
    i = pl.program_id(0)

    @pl.when(i == 0)
    def _():
        pooled[...] = jnp.zeros_like(pooled)
        cnts[...] = jnp.zeros_like(cnts)

    msg = (mp_ref[0] + mp_ref[1])[:, 0:F2]
    dsum = dp_ref[0] + dp_ref[1]
    deg = jnp.maximum(dsum[:, F1:F1 + 1], 1.0)
    pre = msg / deg + jnp.dot(h1_ref[...], r2_ref[...],
                              preferred_element_type=jnp.float32) + b2_ref[...]
    h2 = _elu(pre)
    bb = batch_ref[0, 0, :]
    gids = lax.broadcasted_iota(jnp.int32, (NUM_G, ROWB), 0)
    onehot = (bb[None, :] == gids).astype(jnp.float32)
    pooled[...] += jnp.dot(onehot, h2, preferred_element_type=jnp.float32)
    cnts[...] += jnp.sum(onehot, axis=1, keepdims=True)

    @pl.when(i == NROWB - 1)
    def _():
        pool = pooled[...] / jnp.maximum(cnts[...][:, 0:1], 1.0)
        z = _elu(jnp.dot(pool, f1w_ref[...],
                         preferred_element_type=jnp.float32) + f1b_ref[...])
        o = jnp.dot(z, f2w_ref[...],
                    preferred_element_type=jnp.float32) + f2b_ref[...]
        mx = jnp.max(o, axis=1, keepdims=True)
        o_ref[...] = o - mx - jnp.log(
            jnp.sum(jnp.exp(o - mx), axis=1, keepdims=True))


_tc_c = pl.pallas_call(
    _tc_c_body,
    grid=(NROWB,),
    in_specs=[
        pl.BlockSpec((ROWB, F1), lambda i: (i, 0)),
        pl.BlockSpec((F1, F2), lambda i: (0, 0)),
        pl.BlockSpec((1, F2), lambda i: (0, 0)),
        pl.BlockSpec((NC, ROWB, 128), lambda i: (0, i, 0)),
        pl.BlockSpec((NC, ROWB, 128), lambda i: (0, i, 0)),
        pl.BlockSpec((1, 1, ROWB), lambda i: (i, 0, 0)),
        pl.BlockSpec((F2, 128), lambda i: (0, 0)),
        pl.BlockSpec((1, 128), lambda i: (0, 0)),
        pl.BlockSpec((128, 10), lambda i: (0, 0)),
        pl.BlockSpec((1, 10), lambda i: (0, 0)),
    ],
    out_specs=pl.BlockSpec((NUM_G, 10), lambda i: (0, 0)),
    out_shape=_f32((NUM_G, 10)),
    scratch_shapes=[
        pltpu.VMEM((NUM_G, F2), jnp.float32),
        pltpu.VMEM((NUM_G, 128), jnp.float32),
    ],
)


def kernel(x, edge_index, edge_attr, batch, W1, R1, b1, W2, R2, b2,
           fc1_w, fc1_b, fc2_w, fc2_b):
    src = edge_index[0].reshape(NEB, BLKE)
    dst = edge_index[1].reshape(NEB, BLKE)
    p0 = edge_attr[:, 0].reshape(NEB, BLKE)
    p1 = edge_attr[:, 1].reshape(NEB, BLKE)
    w1big = _pack_taps(W1, (0, 1, 5, 6))
    w2big = _pack_taps(W2, (0, 1))

    y1 = _tc_a(x, w1big).reshape(N * KK, 4 * F1)
    mp1 = _sc_layer1(y1, src, dst, p0, p1)
    h1, y2 = _tc_b(x, R1, b1.reshape(1, F1), mp1, w2big)
    mp2 = _sc_layer2(y2.reshape(N * KK, 2 * F2), src, dst, p0, p1)
    out = _tc_c(h1, R2, b2.reshape(1, F2), mp2, mp1,
                batch.reshape(NROWB, 1, ROWB), fc1_w, fc1_b.reshape(1, 128),
                fc2_w, fc2_b.reshape(1, 10))
    return out


# speedup vs baseline: 8.1696x; 1.0247x over previous
import dataclasses
import functools

import jax
import jax.numpy as jnp
from jax import lax
from jax.experimental import pallas as pl
from jax.experimental.pallas import tpu as pltpu
from jax.experimental.pallas import tpu_sc as plsc

N = 10000
E = 320000
KK = 25
NUM_G = 64
F_IN = 128
F1 = 32
F2 = 64

NC = 2
NS = 16
NTILE = NC * NS
E_PER_TILE = E // NTILE
BLK = 80
NBLK = E_PER_TILE // BLK
RCHUNK = 80
NRCHUNK = N // RCHUNK

_mesh = plsc.VectorSubcoreMesh(core_axis_name="c", subcore_axis_name="s")


def _f32(shape):
    return jax.ShapeDtypeStruct(shape, jnp.float32)


BLKE = 128
NEB = E // BLKE


def _sc_edge_kernel(fo, ngather, with_deg):
    nch = fo // 16

    CH = 64
    scratch = [
        pltpu.VMEM((1, BLKE), jnp.int32),
        pltpu.VMEM((BLKE,), jnp.float32),
        pltpu.VMEM((BLKE,), jnp.float32),
        pltpu.VMEM((BLKE,), jnp.float32),
        pltpu.VMEM((BLKE,), jnp.float32),
        pltpu.VMEM((CH, 128), jnp.float32),
        pltpu.VMEM((CH, 128), jnp.float32),
        pltpu.VMEM(((2 * CH if ngather == 1 else CH), 128),
                   jnp.float32),
        pltpu.VMEM((16, 128), jnp.float32),
        pltpu.VMEM((1, CH), jnp.int32),
        pltpu.VMEM((1, CH), jnp.int32),
        pltpu.VMEM_SHARED((N, 128), jnp.float32),
        pltpu.SemaphoreType.DMA,
        pltpu.SemaphoreType.DMA,
    ]
    if ngather == 2:
        scratch += [
            pltpu.VMEM((1, BLKE), jnp.int32),
            pltpu.VMEM((CH, 128), jnp.float32),
            pltpu.VMEM((CH, 128), jnp.float32),
        ]
    out_type = _f32((NC, N, 128))
    NZCH = N // 16

    def body(y_hbm, src_hbm, dst_hbm, p0_hbm, p1_hbm, msg_out, *refs):
        if ngather == 2:
            (idx0, w0, w1, w2, w3, rowsA0, rowsA1, accr, zbuf, dsth0, dsth1,
             accsh, sem0, sem1, idx1, rowsB0, rowsB1) = refs
        else:
            (idx0, w0, w1, w2, w3, rowsA0, rowsA1, accr, zbuf, dsth0, dsth1,
             accsh, sem0, sem1) = refs
            idx1 = rowsB0 = rowsB1 = None

        c = lax.axis_index("c")
        s = lax.axis_index("s")

        zv = jnp.zeros((16,), jnp.float32)

        @pl.loop(0, 16)
        def _(r):
            for ch in range(8):
                zbuf[r, pl.ds(ch * 16, 16)] = zv

        ov = jnp.ones((16,), jnp.float32)

        @pl.loop(0, 2 * CH if ngather == 1 else CH)
        def _(r):
            for ch in range(8):
                col = ch * 16
                if with_deg and col == F1:
                    accr[r, pl.ds(col, 16)] = ov
                elif col >= fo:
                    accr[r, pl.ds(col, 16)] = zv

        for j in range((NZCH + NS - 1) // NS):
            k = j * NS + s

            @pl.when(k < NZCH)
            def _():
                pltpu.sync_copy(zbuf, accsh.at[pl.ds(k * 16, 16)])

        plsc.subcore_barrier()

        if ngather == 1:
            taps = [(0, 0), (0, fo), (0, 2 * fo), (0, 3 * fo)]
        else:
            taps = [(0, 0), (0, fo), (1, 0), (1, fo)]

        dn = lax.GatherDimensionNumbers(
            offset_dims=(), collapsed_slice_dims=(0,), start_index_map=(0,))

        def _splat(vec, b):
            return lax.gather(
                vec, jnp.full((16, 1), b, jnp.int32), dn, slice_sizes=(1,),
                mode=lax.GatherScatterMode.PROMISE_IN_BOUNDS)

        def _combine(off, bufs, aoff):
            @pl.loop(0, CH // 16)
            def _(g):
                sl = pl.ds(off + g * 16, 16)
                wr = (w0[sl], w1[sl], w2[sl], w3[sl])
                for b in range(16):
                    rb = g * 16 + b
                    svec = tuple(_splat(wr[cc], b) for cc in range(4))
                    for ch in range(nch):
                        acc = None
                        for cc, (bi, coff) in enumerate(taps):
                            term = (bufs[bi][rb, pl.ds(coff + ch * 16, 16)]
                                    * svec[cc])
                            acc = term if acc is None else acc + term
                        accr[aoff + rb, pl.ds(ch * 16, 16)] = acc

        def pbody(src_v, dst_v, p0_v, p1_v):
            for g in range(BLKE // 16):
                sl = pl.ds(g * 16, 16)
                v0 = p0_v[0, sl] * 4.0
                v1 = p1_v[0, sl] * 4.0
                i0 = v0.astype(jnp.int32)
                i1 = v1.astype(jnp.int32)
                fr0 = v0 - i0.astype(jnp.float32)
                fr1 = v1 - i1.astype(jnp.float32)
                q = src_v[0, sl] * KK + i0 + 5 * i1
                idx0[0, sl] = q
                if ngather == 2:
                    idx1[0, sl] = q + 5
                g0 = 1.0 - fr0
                g1 = 1.0 - fr1
                w0[sl] = g0 * g1
                w1[sl] = fr0 * g1
                w2[sl] = g0 * fr1
                w3[sl] = fr0 * fr1

            sems = (sem0, sem1)
            arows = (rowsA0, rowsA1)
            brows = (rowsB0, rowsB1)
            handles = []
            for h in range(2):
                hh = [pltpu.async_copy(
                    y_hbm.at[idx0.at[0, pl.ds(h * CH, CH)]], arows[h],
                    sems[h])]
                if ngather == 2:
                    hh.append(pltpu.async_copy(
                        y_hbm.at[idx1.at[0, pl.ds(h * CH, CH)]], brows[h],
                        sems[h]))
                handles.append(hh)

            dsths = (dsth0, dsth1)
            sc_handles = []
            for h in range(2):
                off = h * CH
                for g2 in range(CH // 16):
                    sl16 = pl.ds(g2 * 16, 16)
                    dsths[h][0, sl16] = dst_v[0, pl.ds(off + g2 * 16, 16)]
                for hdl in handles[h]:
                    hdl.wait()
                if ngather == 1:
                    _combine(off, (arows[h], brows[h]), off)
                    sc_handles.append(pltpu.async_copy(
                        accr.at[pl.ds(off, CH)], accsh.at[dsths[h].at[0]],
                        sems[h], add=True))
                else:
                    _combine(off, (arows[h], brows[h]), 0)
                    pltpu.sync_copy(accr, accsh.at[dsths[h].at[0]], add=True)
            for hdl in sc_handles:
                hdl.wait()

        pltpu.emit_pipeline(
            pbody,
            grid=(NEB,),
            in_specs=[pl.BlockSpec((1, BLKE), lambda i: (i, 0))] * 4,
            out_specs=[],
            core_axis_name=("c", "s"),
            dimension_semantics=(pltpu.PARALLEL,),
        )(src_hbm, dst_hbm, p0_hbm, p1_hbm)

        plsc.subcore_barrier()

        for j in range((NRCHUNK + NS - 1) // NS):
            k = j * NS + s

            @pl.when(k < NRCHUNK)
            def _():
                pltpu.sync_copy(accsh.at[pl.ds(k * RCHUNK, RCHUNK)],
                                msg_out.at[c, pl.ds(k * RCHUNK, RCHUNK)])

    cp = pltpu.CompilerParams(use_tc_tiling_on_sc=False)
    return pl.kernel(body, out_type=out_type, mesh=_mesh,
                     scratch_types=scratch, compiler_params=cp)


_sc_layer1 = _sc_edge_kernel(F1, ngather=1, with_deg=True)
_sc_layer2 = _sc_edge_kernel(F2, ngather=2, with_deg=False)

PACKW = KK * 128


def _pack_taps(W, offsets):
    z = jnp.zeros_like(W[0])
    cols = [W[t + o] if t + o < KK else z
            for t in range(KK) for o in offsets]
    return jnp.concatenate(cols, axis=1)


ROWB = 1000
NROWB = N // ROWB


def _mm_body(x_ref, w_ref, o_ref):
    o_ref[...] = jnp.dot(x_ref[...], w_ref[...],
                         preferred_element_type=jnp.float32)


_tc_a = pl.pallas_call(
    _mm_body,
    grid=(NROWB,),
    in_specs=[
        pl.BlockSpec((ROWB, F_IN), lambda i: (i, 0)),
        pl.BlockSpec((F_IN, PACKW), lambda i: (0, 0)),
    ],
    out_specs=pl.BlockSpec((ROWB, PACKW), lambda i: (i, 0)),
    out_shape=_f32((N, PACKW)),
)


def _elu(v):
    return jnp.where(v > 0, v, jnp.exp(jnp.minimum(v, 0.0)) - 1.0)


def _tc_b_body(x_ref, r1_ref, b1_ref, mp_ref, w2_ref, h1_ref, y2_ref):
    mps = mp_ref[0] + mp_ref[1]
    msg = mps[:, 0:F1]
    deg = jnp.maximum(mps[:, F1:F1 + 1], 1.0)
    pre = msg / deg + jnp.dot(x_ref[...], r1_ref[...],
                              preferred_element_type=jnp.float32) + b1_ref[...]
    h1 = _elu(pre)
    h1_ref[...] = h1
    y2_ref[...] = jnp.dot(h1, w2_ref[...], preferred_element_type=jnp.float32)


_tc_b = pl.pallas_call(
    _tc_b_body,
    grid=(NROWB,),
    in_specs=[
        pl.BlockSpec((ROWB, F_IN), lambda i: (i, 0)),
        pl.BlockSpec((F_IN, F1), lambda i: (0, 0)),
        pl.BlockSpec((1, F1), lambda i: (0, 0)),
        pl.BlockSpec((NC, ROWB, 128), lambda i: (0, i, 0)),
        pl.BlockSpec((F1, PACKW), lambda i: (0, 0)),
    ],
    out_specs=[
        pl.BlockSpec((ROWB, F1), lambda i: (i, 0)),
        pl.BlockSpec((ROWB, PACKW), lambda i: (i, 0)),
    ],
    out_shape=[_f32((N, F1)), _f32((N, PACKW))],
)


def _tc_c_body(h1_ref, r2_ref, b2_ref, mp_ref, dp_ref, batch_ref,
               f1w_ref, f1b_ref, f2w_ref, f2b_ref, o_ref, pooled, cnts):
    i = pl.program_id(0)

    @pl.when(i == 0)
    def _():
        pooled[...] = jnp.zeros_like(pooled)
        cnts[...] = jnp.zeros_like(cnts)

    msg = (mp_ref[0] + mp_ref[1])[:, 0:F2]
    dsum = dp_ref[0] + dp_ref[1]
    deg = jnp.maximum(dsum[:, F1:F1 + 1], 1.0)
    pre = msg / deg + jnp.dot(h1_ref[...], r2_ref[...],
                              preferred_element_type=jnp.float32) + b2_ref[...]
    h2 = _elu(pre)
    bb = batch_ref[0, 0, :]
    gids = lax.broadcasted_iota(jnp.int32, (NUM_G, ROWB), 0)
    onehot = (bb[None, :] == gids).astype(jnp.float32)
    pooled[...] += jnp.dot(onehot, h2, preferred_element_type=jnp.float32)
    cnts[...] += jnp.sum(onehot, axis=1, keepdims=True)

    @pl.when(i == NROWB - 1)
    def _():
        pool = pooled[...] / jnp.maximum(cnts[...][:, 0:1], 1.0)
        z = _elu(jnp.dot(pool, f1w_ref[...],
                         preferred_element_type=jnp.float32) + f1b_ref[...])
        o = jnp.dot(z, f2w_ref[...],
                    preferred_element_type=jnp.float32) + f2b_ref[...]
        mx = jnp.max(o, axis=1, keepdims=True)
        o_ref[...] = o - mx - jnp.log(
            jnp.sum(jnp.exp(o - mx), axis=1, keepdims=True))


_tc_c = pl.pallas_call(
    _tc_c_body,
    grid=(NROWB,),
    in_specs=[
        pl.BlockSpec((ROWB, F1), lambda i: (i, 0)),
        pl.BlockSpec((F1, F2), lambda i: (0, 0)),
        pl.BlockSpec((1, F2), lambda i: (0, 0)),
        pl.BlockSpec((NC, ROWB, 128), lambda i: (0, i, 0)),
        pl.BlockSpec((NC, ROWB, 128), lambda i: (0, i, 0)),
        pl.BlockSpec((1, 1, ROWB), lambda i: (i, 0, 0)),
        pl.BlockSpec((F2, 128), lambda i: (0, 0)),
        pl.BlockSpec((1, 128), lambda i: (0, 0)),
        pl.BlockSpec((128, 10), lambda i: (0, 0)),
        pl.BlockSpec((1, 10), lambda i: (0, 0)),
    ],
    out_specs=pl.BlockSpec((NUM_G, 10), lambda i: (0, 0)),
    out_shape=_f32((NUM_G, 10)),
    scratch_shapes=[
        pltpu.VMEM((NUM_G, F2), jnp.float32),
        pltpu.VMEM((NUM_G, 128), jnp.float32),
    ],
)


def kernel(x, edge_index, edge_attr, batch, W1, R1, b1, W2, R2, b2,
           fc1_w, fc1_b, fc2_w, fc2_b):
    src = edge_index[0].reshape(NEB, BLKE)
    dst = edge_index[1].reshape(NEB, BLKE)
    p0 = edge_attr[:, 0].reshape(NEB, BLKE)
    p1 = edge_attr[:, 1].reshape(NEB, BLKE)
    w1big = _pack_taps(W1, (0, 1, 5, 6))
    w2big = _pack_taps(W2, (0, 1))

    y1 = _tc_a(x, w1big).reshape(N * KK, 4 * F1)
    mp1 = _sc_layer1(y1, src, dst, p0, p1)
    h1, y2 = _tc_b(x, R1, b1.reshape(1, F1), mp1, w2big)
    mp2 = _sc_layer2(y2.reshape(N * KK, 2 * F2), src, dst, p0, p1)
    out = _tc_c(h1, R2, b2.reshape(1, F2), mp2, mp1,
                batch.reshape(NROWB, 1, ROWB), fc1_w, fc1_b.reshape(1, 128),
                fc2_w, fc2_b.reshape(1, 10))
    return out


# speedup vs baseline: 8.1735x; 1.0005x over previous
import jax
import jax.numpy as jnp
from jax import lax
from jax.experimental import pallas as pl
from jax.experimental.pallas import tpu as pltpu
from jax.experimental.pallas import tpu_sc as plsc

N = 10000
E = 320000
KK = 25
NUM_G = 64
F_IN = 128
F1 = 32
F2 = 64

NC = 2
NS = 16
RCHUNK = 80
NRCHUNK = N // RCHUNK

_mesh = plsc.VectorSubcoreMesh(core_axis_name="c", subcore_axis_name="s")


def _f32(shape):
    return jax.ShapeDtypeStruct(shape, jnp.float32)


BLKE = 128
NEB = E // BLKE


def _sc_edge_kernel(fo, ngather, with_deg):
    nch = fo // 16

    CH = 64
    scratch = [
        pltpu.VMEM((1, BLKE), jnp.int32),
        pltpu.VMEM((BLKE,), jnp.float32),
        pltpu.VMEM((BLKE,), jnp.float32),
        pltpu.VMEM((BLKE,), jnp.float32),
        pltpu.VMEM((BLKE,), jnp.float32),
        pltpu.VMEM((CH, 128), jnp.float32),
        pltpu.VMEM((CH, 128), jnp.float32),
        pltpu.VMEM(((2 * CH if ngather == 1 else CH), 128),
                   jnp.float32),
        pltpu.VMEM((16, 128), jnp.float32),
        pltpu.VMEM((1, CH), jnp.int32),
        pltpu.VMEM((1, CH), jnp.int32),
        pltpu.VMEM_SHARED((N, 128), jnp.float32),
        pltpu.SemaphoreType.DMA,
        pltpu.SemaphoreType.DMA,
    ]
    if ngather == 2:
        scratch += [
            pltpu.VMEM((1, BLKE), jnp.int32),
            pltpu.VMEM((CH, 128), jnp.float32),
            pltpu.VMEM((CH, 128), jnp.float32),
        ]
    out_type = _f32((NC, N, 128))
    NZCH = N // 16

    def body(y_hbm, src_hbm, dst_hbm, p0_hbm, p1_hbm, msg_out, *refs):
        if ngather == 2:
            (idx0, w0, w1, w2, w3, rowsA0, rowsA1, accr, zbuf, dsth0, dsth1,
             accsh, sem0, sem1, idx1, rowsB0, rowsB1) = refs
        else:
            (idx0, w0, w1, w2, w3, rowsA0, rowsA1, accr, zbuf, dsth0, dsth1,
             accsh, sem0, sem1) = refs
            idx1 = rowsB0 = rowsB1 = None

        c = lax.axis_index("c")
        s = lax.axis_index("s")

        zv = jnp.zeros((16,), jnp.float32)

        @pl.loop(0, 16)
        def _(r):
            for ch in range(8):
                zbuf[r, pl.ds(ch * 16, 16)] = zv

        ov = jnp.ones((16,), jnp.float32)

        @pl.loop(0, 2 * CH if ngather == 1 else CH)
        def _(r):
            for ch in range(8):
                col = ch * 16
                if with_deg and col == F1:
                    accr[r, pl.ds(col, 16)] = ov
                elif col >= fo:
                    accr[r, pl.ds(col, 16)] = zv

        for j in range((NZCH + NS - 1) // NS):
            k = j * NS + s

            @pl.when(k < NZCH)
            def _():
                pltpu.sync_copy(zbuf, accsh.at[pl.ds(k * 16, 16)])

        plsc.subcore_barrier()

        if ngather == 1:
            taps = [(0, 0), (0, fo), (0, 2 * fo), (0, 3 * fo)]
        else:
            taps = [(0, 0), (0, fo), (1, 0), (1, fo)]

        dn = lax.GatherDimensionNumbers(
            offset_dims=(), collapsed_slice_dims=(0,), start_index_map=(0,))

        def _splat(vec, b):
            return lax.gather(
                vec, jnp.full((16, 1), b, jnp.int32), dn, slice_sizes=(1,),
                mode=lax.GatherScatterMode.PROMISE_IN_BOUNDS)

        def _combine(off, bufs, aoff):
            @pl.loop(0, CH // 16)
            def _(g):
                sl = pl.ds(off + g * 16, 16)
                wr = (w0[sl], w1[sl], w2[sl], w3[sl])
                for b in range(16):
                    rb = g * 16 + b
                    svec = tuple(_splat(wr[cc], b) for cc in range(4))
                    for ch in range(nch):
                        acc = None
                        for cc, (bi, coff) in enumerate(taps):
                            term = (bufs[bi][rb, pl.ds(coff + ch * 16, 16)]
                                    * svec[cc])
                            acc = term if acc is None else acc + term
                        accr[aoff + rb, pl.ds(ch * 16, 16)] = acc

        def pbody(src_v, dst_v, p0_v, p1_v):
            for g in range(BLKE // 16):
                sl = pl.ds(g * 16, 16)
                v0 = p0_v[0, sl] * 4.0
                v1 = p1_v[0, sl] * 4.0
                i0 = v0.astype(jnp.int32)
                i1 = v1.astype(jnp.int32)
                fr0 = v0 - i0.astype(jnp.float32)
                fr1 = v1 - i1.astype(jnp.float32)
                q = src_v[0, sl] * KK + i0 + 5 * i1
                idx0[0, sl] = q
                if ngather == 2:
                    idx1[0, sl] = q + 5
                g0 = 1.0 - fr0
                g1 = 1.0 - fr1
                w0[sl] = g0 * g1
                w1[sl] = fr0 * g1
                w2[sl] = g0 * fr1
                w3[sl] = fr0 * fr1

            sems = (sem0, sem1)
            arows = (rowsA0, rowsA1)
            brows = (rowsB0, rowsB1)
            handles = []
            for h in range(2):
                hh = [pltpu.async_copy(
                    y_hbm.at[idx0.at[0, pl.ds(h * CH, CH)]], arows[h],
                    sems[h])]
                if ngather == 2:
                    hh.append(pltpu.async_copy(
                        y_hbm.at[idx1.at[0, pl.ds(h * CH, CH)]], brows[h],
                        sems[h]))
                handles.append(hh)

            dsths = (dsth0, dsth1)
            sc_handles = []
            for h in range(2):
                off = h * CH
                for g2 in range(CH // 16):
                    sl16 = pl.ds(g2 * 16, 16)
                    dsths[h][0, sl16] = dst_v[0, pl.ds(off + g2 * 16, 16)]
                for hdl in handles[h]:
                    hdl.wait()
                if ngather == 1:
                    _combine(off, (arows[h], brows[h]), off)
                    sc_handles.append(pltpu.async_copy(
                        accr.at[pl.ds(off, CH)], accsh.at[dsths[h].at[0]],
                        sems[h], add=True))
                else:
                    _combine(off, (arows[h], brows[h]), 0)
                    pltpu.sync_copy(accr, accsh.at[dsths[h].at[0]], add=True)
            for hdl in sc_handles:
                hdl.wait()

        pltpu.emit_pipeline(
            pbody,
            grid=(NEB,),
            in_specs=[pl.BlockSpec((1, BLKE), lambda i: (i, 0))] * 4,
            out_specs=[],
            core_axis_name=("c", "s"),
            dimension_semantics=(pltpu.PARALLEL,),
        )(src_hbm, dst_hbm, p0_hbm, p1_hbm)

        plsc.subcore_barrier()

        for j in range((NRCHUNK + NS - 1) // NS):
            k = j * NS + s

            @pl.when(k < NRCHUNK)
            def _():
                pltpu.sync_copy(accsh.at[pl.ds(k * RCHUNK, RCHUNK)],
                                msg_out.at[c, pl.ds(k * RCHUNK, RCHUNK)])

    cp = pltpu.CompilerParams(use_tc_tiling_on_sc=False)
    return pl.kernel(body, out_type=out_type, mesh=_mesh,
                     scratch_types=scratch, compiler_params=cp)


_sc_layer1 = _sc_edge_kernel(F1, ngather=1, with_deg=True)
_sc_layer2 = _sc_edge_kernel(F2, ngather=2, with_deg=False)

PACKW = KK * 128


def _pack_taps(W, offsets):
    z = jnp.zeros_like(W[0])
    cols = [W[t + o] if t + o < KK else z
            for t in range(KK) for o in offsets]
    return jnp.concatenate(cols, axis=1)


ROWB = 1000
NROWB = N // ROWB


def _mm_body(x_ref, w_ref, o_ref):
    o_ref[...] = jnp.dot(x_ref[...], w_ref[...],
                         preferred_element_type=jnp.float32)


_tc_a = pl.pallas_call(
    _mm_body,
    grid=(NROWB,),
    in_specs=[
        pl.BlockSpec((ROWB, F_IN), lambda i: (i, 0)),
        pl.BlockSpec((F_IN, PACKW), lambda i: (0, 0)),
    ],
    out_specs=pl.BlockSpec((ROWB, PACKW), lambda i: (i, 0)),
    out_shape=_f32((N, PACKW)),
)


def _elu(v):
    return jnp.where(v > 0, v, jnp.exp(jnp.minimum(v, 0.0)) - 1.0)


def _tc_b_body(x_ref, r1_ref, b1_ref, mp_ref, w2_ref, h1_ref, y2_ref):
    mps = mp_ref[0] + mp_ref[1]
    msg = mps[:, 0:F1]
    deg = jnp.maximum(mps[:, F1:F1 + 1], 1.0)
    pre = msg / deg + jnp.dot(x_ref[...], r1_ref[...],
                              preferred_element_type=jnp.float32) + b1_ref[...]
    h1 = _elu(pre)
    h1_ref[...] = h1
    y2_ref[...] = jnp.dot(h1, w2_ref[...], preferred_element_type=jnp.float32)


_tc_b = pl.pallas_call(
    _tc_b_body,
    grid=(NROWB,),
    in_specs=[
        pl.BlockSpec((ROWB, F_IN), lambda i: (i, 0)),
        pl.BlockSpec((F_IN, F1), lambda i: (0, 0)),
        pl.BlockSpec((1, F1), lambda i: (0, 0)),
        pl.BlockSpec((NC, ROWB, 128), lambda i: (0, i, 0)),
        pl.BlockSpec((F1, PACKW), lambda i: (0, 0)),
    ],
    out_specs=[
        pl.BlockSpec((ROWB, F1), lambda i: (i, 0)),
        pl.BlockSpec((ROWB, PACKW), lambda i: (i, 0)),
    ],
    out_shape=[_f32((N, F1)), _f32((N, PACKW))],
)


def _tc_c_body(h1_ref, r2_ref, b2_ref, mp_ref, dp_ref, batch_ref,
               f1w_ref, f1b_ref, f2w_ref, f2b_ref, o_ref, pooled, cnts):
    i = pl.program_id(0)

    @pl.when(i == 0)
    def _():
        pooled[...] = jnp.zeros_like(pooled)
        cnts[...] = jnp.zeros_like(cnts)

    msg = (mp_ref[0] + mp_ref[1])[:, 0:F2]
    dsum = dp_ref[0] + dp_ref[1]
    deg = jnp.maximum(dsum[:, F1:F1 + 1], 1.0)
    pre = msg / deg + jnp.dot(h1_ref[...], r2_ref[...],
                              preferred_element_type=jnp.float32) + b2_ref[...]
    h2 = _elu(pre)
    bb = batch_ref[0, 0, :]
    gids = lax.broadcasted_iota(jnp.int32, (NUM_G, ROWB), 0)
    onehot = (bb[None, :] == gids).astype(jnp.float32)
    pooled[...] += jnp.dot(onehot, h2, preferred_element_type=jnp.float32)
    cnts[...] += jnp.sum(onehot, axis=1, keepdims=True)

    @pl.when(i == NROWB - 1)
    def _():
        pool = pooled[...] / jnp.maximum(cnts[...][:, 0:1], 1.0)
        z = _elu(jnp.dot(pool, f1w_ref[...],
                         preferred_element_type=jnp.float32) + f1b_ref[...])
        o = jnp.dot(z, f2w_ref[...],
                    preferred_element_type=jnp.float32) + f2b_ref[...]
        mx = jnp.max(o, axis=1, keepdims=True)
        o_ref[...] = o - mx - jnp.log(
            jnp.sum(jnp.exp(o - mx), axis=1, keepdims=True))


_tc_c = pl.pallas_call(
    _tc_c_body,
    grid=(NROWB,),
    in_specs=[
        pl.BlockSpec((ROWB, F1), lambda i: (i, 0)),
        pl.BlockSpec((F1, F2), lambda i: (0, 0)),
        pl.BlockSpec((1, F2), lambda i: (0, 0)),
        pl.BlockSpec((NC, ROWB, 128), lambda i: (0, i, 0)),
        pl.BlockSpec((NC, ROWB, 128), lambda i: (0, i, 0)),
        pl.BlockSpec((1, 1, ROWB), lambda i: (i, 0, 0)),
        pl.BlockSpec((F2, 128), lambda i: (0, 0)),
        pl.BlockSpec((1, 128), lambda i: (0, 0)),
        pl.BlockSpec((128, 10), lambda i: (0, 0)),
        pl.BlockSpec((1, 10), lambda i: (0, 0)),
    ],
    out_specs=pl.BlockSpec((NUM_G, 10), lambda i: (0, 0)),
    out_shape=_f32((NUM_G, 10)),
    scratch_shapes=[
        pltpu.VMEM((NUM_G, F2), jnp.float32),
        pltpu.VMEM((NUM_G, 128), jnp.float32),
    ],
)


def kernel(x, edge_index, edge_attr, batch, W1, R1, b1, W2, R2, b2,
           fc1_w, fc1_b, fc2_w, fc2_b):
    src = edge_index[0].reshape(NEB, BLKE)
    dst = edge_index[1].reshape(NEB, BLKE)
    p0 = edge_attr[:, 0].reshape(NEB, BLKE)
    p1 = edge_attr[:, 1].reshape(NEB, BLKE)
    w1big = _pack_taps(W1, (0, 1, 5, 6))
    w2big = _pack_taps(W2, (0, 1))

    y1 = _tc_a(x, w1big).reshape(N * KK, 4 * F1)
    mp1 = _sc_layer1(y1, src, dst, p0, p1)
    h1, y2 = _tc_b(x, R1, b1.reshape(1, F1), mp1, w2big)
    mp2 = _sc_layer2(y2.reshape(N * KK, 2 * F2), src, dst, p0, p1)
    out = _tc_c(h1, R2, b2.reshape(1, F2), mp2, mp1,
                batch.reshape(NROWB, 1, ROWB), fc1_w, fc1_b.reshape(1, 128),
                fc2_w, fc2_b.reshape(1, 10))
    return out
